# baseline (speedup 1.0000x reference)
import functools

import jax
import jax.numpy as jnp
from jax import lax
from jax.experimental import pallas as pl
from jax.experimental.pallas import tpu as pltpu
from jax.experimental.pallas import tpu_sc as plsc

B = 4
S = 2048
D = 768
OFFSET = 2
N = B * S

NC, NS = 2, 16
NW = NC * NS
RPW = N // NW
CH = 64
NCH = RPW // CH
PPW = S // NW
PCH = 32
NPCH = PPW // PCH

_mesh = plsc.VectorSubcoreMesh(core_axis_name="c", subcore_axis_name="s")


@functools.partial(
    pl.kernel,
    mesh=_mesh,
    out_type=[
        jax.ShapeDtypeStruct((N, D), jnp.float32),
        jax.ShapeDtypeStruct((B * S * D,), jnp.float32),
    ],
    scratch_types=[
        pltpu.VMEM((NCH, CH), jnp.int32),
        pltpu.VMEM((CH, D), jnp.float32),
        pltpu.VMEM((CH, D), jnp.float32),
        pltpu.VMEM((PCH * D,), jnp.float32),
        pltpu.SemaphoreType.DMA,
        pltpu.SemaphoreType.DMA,
        pltpu.SemaphoreType.DMA,
        pltpu.SemaphoreType.DMA,
        pltpu.SemaphoreType.DMA,
    ],
)
def _emb_kernel(ids_hbm, table_hbm, ptab_hbm, out_tok, out_pos,
                idx_v, buf0, buf1, pos_buf, sem0, sem1, ssem0, ssem1, psem):
    wid = lax.axis_index("s") * NC + lax.axis_index("c")
    base = wid * RPW

    pltpu.sync_copy(ids_hbm.at[pl.ds(wid * NCH, NCH)], idx_v)

    bufs = (buf0, buf1)
    gsems = (sem0, sem1)
    ssems = (ssem0, ssem1)

    def gather(c):
        return pltpu.make_async_copy(
            table_hbm.at[idx_v.at[c]], bufs[c % 2], gsems[c % 2])

    def store(c):
        return pltpu.make_async_copy(
            bufs[c % 2], out_tok.at[pl.ds(base + c * CH, CH)], ssems[c % 2])

    gather(0).start()
    for c in range(NCH):
        if c + 1 < NCH:
            if c >= 1:
                store(c - 1).wait()
            gather(c + 1).start()
        gather(c).wait()
        store(c).start()
    store(NCH - 2).wait()
    store(NCH - 1).wait()

    pbase = wid * PPW
    for p in range(NPCH):
        src = (OFFSET + pbase + p * PCH) * D
        pltpu.sync_copy(ptab_hbm.at[pl.ds(src, PCH * D)], pos_buf)
        for b in range(B):
            dst = (b * S + pbase + p * PCH) * D
            pltpu.make_async_copy(
                pos_buf, out_pos.at[pl.ds(dst, PCH * D)], psem).start()
        for b in range(B):
            pltpu.make_async_copy(
                pos_buf, out_pos.at[pl.ds((b * S + pbase + p * PCH) * D,
                                          PCH * D)], psem).wait()


def kernel(input_ids, embed_tokens, embed_positions):
    ids = input_ids.reshape(NW * NCH, CH).astype(jnp.int32)
    tok_flat, pos_flat = _emb_kernel(ids, embed_tokens,
                                     embed_positions.reshape(-1))
    inputs_embeds = tok_flat.reshape(B, S, D)
    pos_embeds = pos_flat.reshape(B, S, D)
    attention_mask = jnp.ones((B, S), dtype=jnp.float32)
    return (inputs_embeds, attention_mask, pos_embeds)


# speedup vs baseline: 1.6353x; 1.6353x over previous
import functools

import jax
import jax.numpy as jnp
from jax import lax
from jax.experimental import pallas as pl
from jax.experimental.pallas import tpu as pltpu
from jax.experimental.pallas import tpu_sc as plsc

B = 4
S = 2048
D = 768
OFFSET = 2
N = B * S

NC, NS = 2, 16
NW = NC * NS
RPW = N // NW
CH = 64
NCH = RPW // CH

_mesh = plsc.VectorSubcoreMesh(core_axis_name="c", subcore_axis_name="s")


@functools.partial(
    pl.kernel,
    mesh=_mesh,
    out_type=jax.ShapeDtypeStruct((N, D), jnp.float32),
    scratch_types=[
        pltpu.VMEM((NCH, CH), jnp.int32),
        pltpu.VMEM((CH, D), jnp.float32),
        pltpu.VMEM((CH, D), jnp.float32),
        pltpu.SemaphoreType.DMA,
        pltpu.SemaphoreType.DMA,
        pltpu.SemaphoreType.DMA,
        pltpu.SemaphoreType.DMA,
    ],
)
def _tok_kernel(ids_hbm, table_hbm, out_tok,
                idx_v, buf0, buf1, sem0, sem1, ssem0, ssem1):
    wid = lax.axis_index("s") * NC + lax.axis_index("c")
    base = wid * RPW

    pltpu.sync_copy(ids_hbm.at[pl.ds(wid * NCH, NCH)], idx_v)

    bufs = (buf0, buf1)
    gsems = (sem0, sem1)
    ssems = (ssem0, ssem1)

    def gather(c):
        return pltpu.make_async_copy(
            table_hbm.at[idx_v.at[c]], bufs[c % 2], gsems[c % 2])

    def store(c):
        return pltpu.make_async_copy(
            bufs[c % 2], out_tok.at[pl.ds(base + c * CH, CH)], ssems[c % 2])

    gather(0).start()
    for c in range(NCH):
        if c + 1 < NCH:
            if c >= 1:
                store(c - 1).wait()
            gather(c + 1).start()
        gather(c).wait()
        store(c).start()
    store(NCH - 2).wait()
    store(NCH - 1).wait()


PBS = 512


def _pos_body(src_ref, out_ref):
    out_ref[0] = src_ref[...]


_pos_kernel = pl.pallas_call(
    _pos_body,
    grid=(S // PBS, B),
    in_specs=[pl.BlockSpec((PBS, D), lambda j, b: (j, 0))],
    out_specs=pl.BlockSpec((1, PBS, D), lambda j, b: (b, j, 0)),
    out_shape=jax.ShapeDtypeStruct((B, S, D), jnp.float32),
)


def kernel(input_ids, embed_tokens, embed_positions):
    ids = input_ids.reshape(NW * NCH, CH).astype(jnp.int32)
    tok_flat = _tok_kernel(ids, embed_tokens)
    pos_slice = lax.slice_in_dim(embed_positions, OFFSET, OFFSET + S)
    pos_embeds = _pos_kernel(pos_slice)
    inputs_embeds = tok_flat.reshape(B, S, D)
    attention_mask = jnp.ones((B, S), dtype=jnp.float32)
    return (inputs_embeds, attention_mask, pos_embeds)


# speedup vs baseline: 1.7590x; 1.0756x over previous
import functools

import jax
import jax.numpy as jnp
from jax import lax
from jax.experimental import pallas as pl
from jax.experimental.pallas import tpu as pltpu
from jax.experimental.pallas import tpu_sc as plsc

B = 4
S = 2048
D = 768
MAX_POS = 2048
OFFSET = 2
N = B * S

NC, NS = 2, 16
NW = NC * NS
RPW = N // NW
CH = 64
NCH = RPW // CH

_mesh = plsc.VectorSubcoreMesh(core_axis_name="c", subcore_axis_name="s")


@functools.partial(
    pl.kernel,
    mesh=_mesh,
    out_type=jax.ShapeDtypeStruct((N, D), jnp.float32),
    scratch_types=[
        pltpu.VMEM((NCH, CH), jnp.int32),
        pltpu.VMEM((CH, D), jnp.float32),
        pltpu.VMEM((CH, D), jnp.float32),
        pltpu.SemaphoreType.DMA,
        pltpu.SemaphoreType.DMA,
        pltpu.SemaphoreType.DMA,
        pltpu.SemaphoreType.DMA,
    ],
)
def _tok_kernel(ids_hbm, table_hbm, out_tok,
                idx_v, buf0, buf1, sem0, sem1, ssem0, ssem1):
    wid = lax.axis_index("s") * NC + lax.axis_index("c")
    base = wid * RPW

    pltpu.sync_copy(ids_hbm.at[pl.ds(wid * NCH, NCH)], idx_v)

    bufs = (buf0, buf1)
    gsems = (sem0, sem1)
    ssems = (ssem0, ssem1)

    def gather(c):
        return pltpu.make_async_copy(
            table_hbm.at[idx_v.at[c]], bufs[c % 2], gsems[c % 2])

    def store(c):
        return pltpu.make_async_copy(
            bufs[c % 2], out_tok.at[pl.ds(base + c * CH, CH)], ssems[c % 2])

    gather(0).start()
    for c in range(NCH):
        if c + 1 < NCH:
            if c >= 1:
                store(c - 1).wait()
            gather(c + 1).start()
        gather(c).wait()
        store(c).start()
    store(NCH - 2).wait()
    store(NCH - 1).wait()


def _pos_body(src_ref, out_ref):
    out_ref[0] = src_ref[pl.ds(OFFSET, S), :]


_pos_kernel = pl.pallas_call(
    _pos_body,
    grid=(B,),
    in_specs=[pl.BlockSpec((MAX_POS + OFFSET, D), lambda b: (0, 0))],
    out_specs=pl.BlockSpec((1, S, D), lambda b: (b, 0, 0)),
    out_shape=jax.ShapeDtypeStruct((B, S, D), jnp.float32),
)


def kernel(input_ids, embed_tokens, embed_positions):
    ids = input_ids.reshape(NW * NCH, CH).astype(jnp.int32)
    tok_flat = _tok_kernel(ids, embed_tokens)
    pos_embeds = _pos_kernel(embed_positions)
    inputs_embeds = tok_flat.reshape(B, S, D)
    attention_mask = jnp.ones((B, S), dtype=jnp.float32)
    return (inputs_embeds, attention_mask, pos_embeds)


# speedup vs baseline: 1.7946x; 1.0202x over previous
import functools

import jax
import jax.numpy as jnp
from jax import lax
from jax.experimental import pallas as pl
from jax.experimental.pallas import tpu as pltpu
from jax.experimental.pallas import tpu_sc as plsc

B = 4
S = 2048
D = 768
MAX_POS = 2048
OFFSET = 2
N = B * S

NC, NS = 2, 16
NW = NC * NS
RPW = N // NW
CH = 64
NCH = RPW // CH

_mesh = plsc.VectorSubcoreMesh(core_axis_name="c", subcore_axis_name="s")


L = 16


@functools.partial(
    pl.kernel,
    mesh=_mesh,
    out_type=jax.ShapeDtypeStruct((N, D), jnp.float32),
    scratch_types=[
        pltpu.VMEM((B, RPW), jnp.int32),
        pltpu.VMEM((CH, D), jnp.float32),
        pltpu.VMEM((CH, D), jnp.float32),
        pltpu.SemaphoreType.DMA,
        pltpu.SemaphoreType.DMA,
        pltpu.SemaphoreType.DMA,
        pltpu.SemaphoreType.DMA,
    ],
)
def _tok_kernel(ids_hbm, table_hbm, out_tok,
                idx_v, buf0, buf1, sem0, sem1, ssem0, ssem1):
    wid = lax.axis_index("s") * NC + lax.axis_index("c")
    b = wid % B
    cb = wid // B
    base = b * S + cb * RPW

    pltpu.sync_copy(ids_hbm.at[:, pl.ds(cb * RPW, RPW)], idx_v)

    bufs = (buf0, buf1)
    gsems = (sem0, sem1)
    ssems = (ssem0, ssem1)

    def gather(c):
        cps = []
        for k in range(CH // L):
            vec = idx_v[b, pl.ds(c * CH + k * L, L)]
            cp = pltpu.make_async_copy(
                table_hbm.at[vec],
                bufs[c % 2].at[pl.ds(k * L, L)],
                gsems[c % 2],
            )
            cp.start()
            cps.append(cp)
        return cps

    def store(c):
        return pltpu.make_async_copy(
            bufs[c % 2], out_tok.at[pl.ds(base + c * CH, CH)], ssems[c % 2])

    pend = {0: gather(0)}
    for c in range(NCH):
        if c + 1 < NCH:
            if c >= 1:
                store(c - 1).wait()
            pend[c + 1] = gather(c + 1)
        for cp in pend.pop(c):
            cp.wait()
        store(c).start()
    store(NCH - 2).wait()
    store(NCH - 1).wait()


def _pos_body(src_ref, out_ref):
    out_ref[0] = src_ref[pl.ds(OFFSET, S), :]


_pos_kernel = pl.pallas_call(
    _pos_body,
    grid=(B,),
    in_specs=[pl.BlockSpec((MAX_POS + OFFSET, D), lambda b: (0, 0))],
    out_specs=pl.BlockSpec((1, S, D), lambda b: (b, 0, 0)),
    out_shape=jax.ShapeDtypeStruct((B, S, D), jnp.float32),
)


def kernel(input_ids, embed_tokens, embed_positions):
    ids = input_ids.astype(jnp.int32)
    tok_flat = _tok_kernel(ids, embed_tokens)
    pos_embeds = _pos_kernel(embed_positions)
    inputs_embeds = tok_flat.reshape(B, S, D)
    attention_mask = jnp.ones((B, S), dtype=jnp.float32)
    return (inputs_embeds, attention_mask, pos_embeds)


# speedup vs baseline: 1.8188x; 1.0135x over previous
import functools

import jax
import jax.numpy as jnp
from jax import lax
from jax.experimental import pallas as pl
from jax.experimental.pallas import tpu as pltpu
from jax.experimental.pallas import tpu_sc as plsc

B = 4
S = 2048
D = 768
MAX_POS = 2048
OFFSET = 2
N = B * S

NC, NS = 2, 16
NW = NC * NS
RPW = N // NW
CH = 32
NCH = RPW // CH
NBUF = 5

_mesh = plsc.VectorSubcoreMesh(core_axis_name="c", subcore_axis_name="s")


L = 16


@functools.partial(
    pl.kernel,
    mesh=_mesh,
    out_type=jax.ShapeDtypeStruct((N, D), jnp.float32),
    scratch_types=[
        pltpu.VMEM((B, RPW), jnp.int32),
    ]
    + [pltpu.VMEM((CH, D), jnp.float32) for _ in range(NBUF)]
    + [pltpu.SemaphoreType.DMA for _ in range(2 * NBUF)],
)
def _tok_kernel(ids_hbm, table_hbm, out_tok, idx_v, *bufs_and_sems):
    bufs = bufs_and_sems[:NBUF]
    gsems = bufs_and_sems[NBUF:2 * NBUF]
    ssems = bufs_and_sems[2 * NBUF:]
    wid = lax.axis_index("s") * NC + lax.axis_index("c")
    b = wid % B
    cb = wid // B
    base = b * S + cb * RPW

    pltpu.sync_copy(ids_hbm.at[:, pl.ds(cb * RPW, RPW)], idx_v)

    def gather(c):
        cps = []
        for k in range(CH // L):
            vec = idx_v[b, pl.ds(c * CH + k * L, L)]
            cp = pltpu.make_async_copy(
                table_hbm.at[vec],
                bufs[c % NBUF].at[pl.ds(k * L, L)],
                gsems[c % NBUF],
            )
            cp.start()
            cps.append(cp)
        return cps

    def store(c):
        return pltpu.make_async_copy(
            bufs[c % NBUF], out_tok.at[pl.ds(base + c * CH, CH)],
            ssems[c % NBUF])

    LA = 3
    pend = {c: gather(c) for c in range(min(LA, NCH))}
    stores = {}
    for c in range(NCH):
        for cp in pend.pop(c):
            cp.wait()
        stores[c] = store(c)
        stores[c].start()
        nxt = c + LA
        if nxt < NCH:
            old = nxt - NBUF
            if old >= 0:
                stores.pop(old).wait()
            pend[nxt] = gather(nxt)
    for c in sorted(stores):
        stores[c].wait()


def _pos_body(src_ref, out_ref):
    out_ref[0] = src_ref[pl.ds(OFFSET, S), :]


_pos_kernel = pl.pallas_call(
    _pos_body,
    grid=(B,),
    in_specs=[pl.BlockSpec((MAX_POS + OFFSET, D), lambda b: (0, 0))],
    out_specs=pl.BlockSpec((1, S, D), lambda b: (b, 0, 0)),
    out_shape=jax.ShapeDtypeStruct((B, S, D), jnp.float32),
)


def kernel(input_ids, embed_tokens, embed_positions):
    ids = input_ids.astype(jnp.int32)
    tok_flat = _tok_kernel(ids, embed_tokens)
    pos_embeds = _pos_kernel(embed_positions)
    inputs_embeds = tok_flat.reshape(B, S, D)
    attention_mask = jnp.ones((B, S), dtype=jnp.float32)
    return (inputs_embeds, attention_mask, pos_embeds)


# speedup vs baseline: 1.8449x; 1.0143x over previous
import functools

import jax
import jax.numpy as jnp
from jax import lax
from jax.experimental import pallas as pl
from jax.experimental.pallas import tpu as pltpu
from jax.experimental.pallas import tpu_sc as plsc

B = 4
S = 2048
D = 768
MAX_POS = 2048
OFFSET = 2
N = B * S

NC, NS = 2, 16
NW = NC * NS
RPW = N // NW
CH = 32
NCH = RPW // CH
NBUF = 5

_mesh = plsc.VectorSubcoreMesh(core_axis_name="c", subcore_axis_name="s")


L = 16


@functools.partial(
    pl.kernel,
    mesh=_mesh,
    out_type=jax.ShapeDtypeStruct((N, D), jnp.float32),
    scratch_types=[
        pltpu.VMEM((B, RPW), jnp.int32),
    ]
    + [pltpu.VMEM((CH, D), jnp.float32) for _ in range(NBUF)]
    + [pltpu.SemaphoreType.DMA for _ in range(2 * NBUF)],
)
def _tok_kernel(ids_hbm, table_hbm, out_tok, idx_v, *bufs_and_sems):
    bufs = bufs_and_sems[:NBUF]
    gsems = bufs_and_sems[NBUF:2 * NBUF]
    ssems = bufs_and_sems[2 * NBUF:]
    wid = lax.axis_index("s") * NC + lax.axis_index("c")
    b = wid % B
    cb = wid // B
    base = b * S + cb * RPW

    pltpu.sync_copy(ids_hbm.at[:, pl.ds(cb * RPW, RPW)], idx_v)

    def gather(c):
        cps = []
        for k in range(CH // L):
            vec = idx_v[b, pl.ds(c * CH + k * L, L)]
            cp = pltpu.make_async_copy(
                table_hbm.at[vec],
                bufs[c % NBUF].at[pl.ds(k * L, L)],
                gsems[c % NBUF],
            )
            cp.start()
            cps.append(cp)
        return cps

    def store(c):
        return pltpu.make_async_copy(
            bufs[c % NBUF], out_tok.at[pl.ds(base + c * CH, CH)],
            ssems[c % NBUF])

    LA = 3
    pend = {c: gather(c) for c in range(min(LA, NCH))}
    stores = {}
    for c in range(NCH):
        for cp in pend.pop(c):
            cp.wait()
        stores[c] = store(c)
        stores[c].start()
        nxt = c + LA
        if nxt < NCH:
            old = nxt - NBUF
            if old >= 0:
                stores.pop(old).wait()
            pend[nxt] = gather(nxt)
    for c in sorted(stores):
        stores[c].wait()


def _pos_body(src_ref, out_ref, mask_ref):
    out_ref[0] = src_ref[pl.ds(OFFSET, S), :]

    @pl.when(pl.program_id(0) == 0)
    def _():
        mask_ref[...] = jnp.ones_like(mask_ref)


_pos_kernel = pl.pallas_call(
    _pos_body,
    grid=(B,),
    in_specs=[pl.BlockSpec((MAX_POS + OFFSET, D), lambda b: (0, 0))],
    out_specs=[
        pl.BlockSpec((1, S, D), lambda b: (b, 0, 0)),
        pl.BlockSpec((B, S), lambda b: (0, 0)),
    ],
    out_shape=[
        jax.ShapeDtypeStruct((B, S, D), jnp.float32),
        jax.ShapeDtypeStruct((B, S), jnp.float32),
    ],
)


def kernel(input_ids, embed_tokens, embed_positions):
    ids = input_ids.astype(jnp.int32)
    tok_flat = _tok_kernel(ids, embed_tokens)
    pos_embeds, attention_mask = _pos_kernel(embed_positions)
    inputs_embeds = tok_flat.reshape(B, S, D)
    return (inputs_embeds, attention_mask, pos_embeds)
